# Initial kernel scaffold; baseline (speedup 1.0000x reference)
import jax
import jax.numpy as jnp
from jax.experimental import pallas as pl


def kernel(p, f, pe, knn_idx, W1, gamma1, beta1, W2, gamma2, beta2):
    raise NotImplementedError("write your pallas kernel here")


# speedup vs baseline: 1.9587x; 1.9587x over previous
import functools

import jax
import jax.numpy as jnp
from jax.experimental import pallas as pl
from jax.experimental.pallas import tpu as pltpu

B, N, K = 4, 1024, 8
C0, C1, C2 = 1024, 2048, 2048
EPS = 1e-5

NB_MM = 256
NB_G = 128


def _mm_stats_body(x_ref, w_ref, y_ref, stats_ref, *, act, scale_ref=None,
                   bias_ref=None):
    x = x_ref[0]
    if act:
        x = jnp.maximum(x * scale_ref[0][None, :] + bias_ref[0][None, :], 0.0)
    y = jnp.dot(x, w_ref[...], preferred_element_type=jnp.float32)
    y_ref[0] = y

    @pl.when((pl.program_id(0) == 0) & (pl.program_id(1) == 0))
    def _():
        stats_ref[...] = jnp.zeros_like(stats_ref)

    stats_ref[...] += jnp.concatenate(
        [jnp.sum(y, axis=0)[None, :], jnp.sum(y * y, axis=0)[None, :]], axis=0)


def _mm1_kernel(x_ref, w_ref, y_ref, stats_ref):
    _mm_stats_body(x_ref, w_ref, y_ref, stats_ref, act=False)


def _mm2_kernel(x_ref, s_ref, b_ref, w_ref, y_ref, stats_ref):
    _mm_stats_body(x_ref, w_ref, y_ref, stats_ref, act=True,
                   scale_ref=s_ref, bias_ref=b_ref)


def _mm1(xT, wT):
    cin, cout = wT.shape
    return pl.pallas_call(
        _mm1_kernel,
        grid=(B, N // NB_MM),
        in_specs=[
            pl.BlockSpec((1, NB_MM, cin), lambda b, n: (b, n, 0)),
            pl.BlockSpec((cin, cout), lambda b, n: (0, 0)),
        ],
        out_specs=[
            pl.BlockSpec((1, NB_MM, cout), lambda b, n: (b, n, 0)),
            pl.BlockSpec((2, cout), lambda b, n: (0, 0)),
        ],
        out_shape=[
            jax.ShapeDtypeStruct((B, N, cout), jnp.float32),
            jax.ShapeDtypeStruct((2, cout), jnp.float32),
        ],
    )(xT, wT)


def _mm2(xT, scale, bias, wT):
    cin, cout = wT.shape

    def kern(x_ref, s_ref, b_ref, w_ref, y_ref, stats_ref):
        x = jnp.maximum(x_ref[0] * s_ref[0][None, :] + b_ref[0][None, :], 0.0)
        y = jnp.dot(x, w_ref[...], preferred_element_type=jnp.float32)
        y_ref[0] = y.reshape(NB_MM, 16, 128)

        @pl.when((pl.program_id(0) == 0) & (pl.program_id(1) == 0))
        def _():
            stats_ref[...] = jnp.zeros_like(stats_ref)

        stats_ref[...] += jnp.concatenate(
            [jnp.sum(y, axis=0)[None, :], jnp.sum(y * y, axis=0)[None, :]],
            axis=0)

    return pl.pallas_call(
        kern,
        grid=(B, N // NB_MM),
        in_specs=[
            pl.BlockSpec((1, NB_MM, cin), lambda b, n: (b, n, 0)),
            pl.BlockSpec((1, cin), lambda b, n: (0, 0)),
            pl.BlockSpec((1, cin), lambda b, n: (0, 0)),
            pl.BlockSpec((cin, cout), lambda b, n: (0, 0)),
        ],
        out_specs=[
            pl.BlockSpec((1, NB_MM, 16, 128), lambda b, n: (b, n, 0, 0)),
            pl.BlockSpec((2, cout), lambda b, n: (0, 0)),
        ],
        out_shape=[
            jax.ShapeDtypeStruct((B, N, 16, 128), jnp.float32),
            jax.ShapeDtypeStruct((2, cout), jnp.float32),
        ],
    )(xT, scale[None, :], bias[None, :], wT)


def _norm_kernel(y_ref, s_ref, b_ref, o_ref):
    o_ref[0] = jnp.maximum(y_ref[0] * s_ref[0][None] + b_ref[0][None], 0.0)


def _normalize(y, scale, bias):
    return pl.pallas_call(
        _norm_kernel,
        grid=(B,),
        in_specs=[
            pl.BlockSpec((1, N, 16, 128), lambda b: (b, 0, 0, 0)),
            pl.BlockSpec((1, 16, 128), lambda b: (0, 0, 0)),
            pl.BlockSpec((1, 16, 128), lambda b: (0, 0, 0)),
        ],
        out_specs=pl.BlockSpec((1, N, 16, 128), lambda b: (b, 0, 0, 0)),
        out_shape=jax.ShapeDtypeStruct((B, N, 16, 128), jnp.float32),
    )(y, scale[None], bias[None])


def _gather_max_kernel(knn_sm, y_ref, pe_ref, o_ref, g_ref):
    b = pl.program_id(0)
    nb = pl.program_id(1)
    base = b * (N * K) + nb * (NB_G * K)

    def body(j, carry):
        idx = knn_sm[base + j]
        g_ref[j] = y_ref[0, idx]
        return carry

    jax.lax.fori_loop(0, NB_G * K, body, 0, unroll=8)

    g = g_ref[...].reshape(NB_G * K, C2)
    s = g + pe_ref[0].T
    m = jnp.max(s.reshape(NB_G, K, C2), axis=1)
    o_ref[0] = m.T


def _gather_max(y2n, pe_r, knn_flat):
    grid_spec = pltpu.PrefetchScalarGridSpec(
        num_scalar_prefetch=1,
        grid=(B, N // NB_G),
        in_specs=[
            pl.BlockSpec((1, N, 16, 128), lambda b, n, knn: (b, 0, 0, 0)),
            pl.BlockSpec((1, C2, NB_G * K), lambda b, n, knn: (b, 0, n)),
        ],
        out_specs=pl.BlockSpec((1, C2, NB_G), lambda b, n, knn: (b, 0, n)),
        scratch_shapes=[pltpu.VMEM((NB_G * K, 16, 128), jnp.float32)],
    )
    return pl.pallas_call(
        _gather_max_kernel,
        grid_spec=grid_spec,
        out_shape=jax.ShapeDtypeStruct((B, C2, N), jnp.float32),
    )(knn_flat, y2n, pe_r)


def _bn_coeffs(stats, gamma, beta):
    cnt = float(B * N)
    mean = stats[0] / cnt
    var = stats[1] / cnt - mean * mean
    scale = gamma * jax.lax.rsqrt(var + EPS)
    bias = beta - mean * scale
    return scale, bias


def kernel(p, f, pe, knn_idx, W1, gamma1, beta1, W2, gamma2, beta2):
    fT = jnp.transpose(f, (0, 2, 1))
    y1, st1 = _mm1(fT, W1.T)
    s1, b1 = _bn_coeffs(st1, gamma1, beta1)
    y2, st2 = _mm2(y1, s1, b1, W2.T)
    s2, b2 = _bn_coeffs(st2, gamma2, beta2)
    y2n = _normalize(y2, s2.reshape(16, 128), b2.reshape(16, 128))
    out = _gather_max(y2n, pe.reshape(B, C2, N * K), knn_idx.reshape(-1))
    return (out, knn_idx)


# speedup vs baseline: 3.7780x; 1.9288x over previous
import functools

import jax
import jax.numpy as jnp
from jax.experimental import pallas as pl
from jax.experimental.pallas import tpu as pltpu

B, N, K = 4, 1024, 8
C0, C1, C2 = 1024, 2048, 2048
EPS = 1e-5

NB_MM = 256
NB_G = 128


def _mm1_kernel(x_ref, w_ref, y_ref, stats_ref):
    y = jax.lax.dot_general(x_ref[0], w_ref[...], (((0,), (0,)), ((), ())),
                            preferred_element_type=jnp.float32)
    y_ref[0] = y

    @pl.when((pl.program_id(0) == 0) & (pl.program_id(1) == 0))
    def _():
        stats_ref[...] = jnp.zeros_like(stats_ref)

    stats_ref[...] += jnp.concatenate(
        [jnp.sum(y, axis=0)[None, :], jnp.sum(y * y, axis=0)[None, :]], axis=0)


def _mm1(x, wT):
    cin, cout = wT.shape
    return pl.pallas_call(
        _mm1_kernel,
        grid=(B, N // NB_MM),
        in_specs=[
            pl.BlockSpec((1, cin, NB_MM), lambda b, n: (b, 0, n)),
            pl.BlockSpec((cin, cout), lambda b, n: (0, 0)),
        ],
        out_specs=[
            pl.BlockSpec((1, NB_MM, cout), lambda b, n: (b, n, 0)),
            pl.BlockSpec((2, cout), lambda b, n: (0, 0)),
        ],
        out_shape=[
            jax.ShapeDtypeStruct((B, N, cout), jnp.float32),
            jax.ShapeDtypeStruct((2, cout), jnp.float32),
        ],
    )(x, wT)


def _mm2(xT, scale, bias, wT):
    cin, cout = wT.shape

    def kern(x_ref, s_ref, b_ref, w_ref, y_ref, stats_ref):
        x = jnp.maximum(x_ref[0] * s_ref[0][None, :] + b_ref[0][None, :], 0.0)
        y = jnp.dot(x, w_ref[...], preferred_element_type=jnp.float32)
        y_ref[0] = y.reshape(NB_MM, 16, 128)

        @pl.when((pl.program_id(0) == 0) & (pl.program_id(1) == 0))
        def _():
            stats_ref[...] = jnp.zeros_like(stats_ref)

        stats_ref[...] += jnp.concatenate(
            [jnp.sum(y, axis=0)[None, :], jnp.sum(y * y, axis=0)[None, :]],
            axis=0)

    return pl.pallas_call(
        kern,
        grid=(B, N // NB_MM),
        in_specs=[
            pl.BlockSpec((1, NB_MM, cin), lambda b, n: (b, n, 0)),
            pl.BlockSpec((1, cin), lambda b, n: (0, 0)),
            pl.BlockSpec((1, cin), lambda b, n: (0, 0)),
            pl.BlockSpec((cin, cout), lambda b, n: (0, 0)),
        ],
        out_specs=[
            pl.BlockSpec((1, NB_MM, 16, 128), lambda b, n: (b, n, 0, 0)),
            pl.BlockSpec((2, cout), lambda b, n: (0, 0)),
        ],
        out_shape=[
            jax.ShapeDtypeStruct((B, N, 16, 128), jnp.float32),
            jax.ShapeDtypeStruct((2, cout), jnp.float32),
        ],
    )(xT, scale[None, :], bias[None, :], wT)


def _norm_kernel(y_ref, s_ref, b_ref, o_ref):
    o_ref[0] = jnp.maximum(y_ref[0] * s_ref[0][None] + b_ref[0][None], 0.0)


def _normalize(y, scale, bias):
    return pl.pallas_call(
        _norm_kernel,
        grid=(B,),
        in_specs=[
            pl.BlockSpec((1, N, 16, 128), lambda b: (b, 0, 0, 0)),
            pl.BlockSpec((1, 16, 128), lambda b: (0, 0, 0)),
            pl.BlockSpec((1, 16, 128), lambda b: (0, 0, 0)),
        ],
        out_specs=pl.BlockSpec((1, N, 16, 128), lambda b: (b, 0, 0, 0)),
        out_shape=jax.ShapeDtypeStruct((B, N, 16, 128), jnp.float32),
    )(y, scale[None], bias[None])


def _gather_max_kernel(knn_sm, y_ref, pe_ref, o_ref, g_ref):
    b = pl.program_id(0)
    nb = pl.program_id(1)
    base = b * (N * K) + nb * (NB_G * K)

    def body(j, carry):
        n = j & (NB_G - 1)
        k = j >> 7
        idx = knn_sm[base + n * K + k]
        g_ref[j] = y_ref[0, idx]
        return carry

    jax.lax.fori_loop(0, NB_G * K, body, 0, unroll=8)

    g2 = g_ref[...].reshape(NB_G * K, C2)
    s = g2.T + pe_ref[0].reshape(C2, K * NB_G)
    m = s[:, 0:NB_G]
    for k in range(1, K):
        m = jnp.maximum(m, s[:, k * NB_G:(k + 1) * NB_G])
    o_ref[0] = m


def _gather_max(y2n, pe_kn, knn_flat):
    grid_spec = pltpu.PrefetchScalarGridSpec(
        num_scalar_prefetch=1,
        grid=(B, N // NB_G),
        in_specs=[
            pl.BlockSpec((1, N, 16, 128), lambda b, n, knn: (b, 0, 0, 0)),
            pl.BlockSpec((1, C2, K, NB_G), lambda b, n, knn: (b, 0, 0, n)),
        ],
        out_specs=pl.BlockSpec((1, C2, NB_G), lambda b, n, knn: (b, 0, n)),
        scratch_shapes=[pltpu.VMEM((NB_G * K, 16, 128), jnp.float32)],
    )
    return pl.pallas_call(
        _gather_max_kernel,
        grid_spec=grid_spec,
        out_shape=jax.ShapeDtypeStruct((B, C2, N), jnp.float32),
    )(knn_flat, y2n, pe_kn)


def _bn_coeffs(stats, gamma, beta):
    cnt = float(B * N)
    mean = stats[0] / cnt
    var = stats[1] / cnt - mean * mean
    scale = gamma * jax.lax.rsqrt(var + EPS)
    bias = beta - mean * scale
    return scale, bias


def kernel(p, f, pe, knn_idx, W1, gamma1, beta1, W2, gamma2, beta2):
    y1, st1 = _mm1(f, W1.T)
    s1, b1 = _bn_coeffs(st1, gamma1, beta1)
    y2, st2 = _mm2(y1, s1, b1, W2.T)
    s2, b2 = _bn_coeffs(st2, gamma2, beta2)
    y2n = _normalize(y2, s2.reshape(16, 128), b2.reshape(16, 128))
    pe_kn = jnp.transpose(pe, (0, 1, 3, 2))
    out = _gather_max(y2n, pe_kn, knn_idx.reshape(-1))
    return (out, knn_idx)


# speedup vs baseline: 4.1333x; 1.0940x over previous
import functools

import jax
import jax.numpy as jnp
from jax.experimental import pallas as pl
from jax.experimental.pallas import tpu as pltpu

B, N, K = 4, 1024, 8
C0, C1, C2 = 1024, 2048, 2048
EPS = 1e-5

NB_MM = 256
NB_G = 128


def _mm1_kernel(x_ref, w_ref, y_ref, stats_ref):
    y = jax.lax.dot_general(x_ref[0], w_ref[...], (((0,), (0,)), ((), ())),
                            preferred_element_type=jnp.float32)
    y_ref[0] = y.astype(jnp.bfloat16)

    @pl.when((pl.program_id(0) == 0) & (pl.program_id(1) == 0))
    def _():
        stats_ref[...] = jnp.zeros_like(stats_ref)

    stats_ref[...] += jnp.concatenate(
        [jnp.sum(y, axis=0)[None, :], jnp.sum(y * y, axis=0)[None, :]], axis=0)


def _mm1(x, wT):
    cin, cout = wT.shape
    return pl.pallas_call(
        _mm1_kernel,
        grid=(B, N // NB_MM),
        in_specs=[
            pl.BlockSpec((1, cin, NB_MM), lambda b, n: (b, 0, n)),
            pl.BlockSpec((cin, cout), lambda b, n: (0, 0)),
        ],
        out_specs=[
            pl.BlockSpec((1, NB_MM, cout), lambda b, n: (b, n, 0)),
            pl.BlockSpec((2, cout), lambda b, n: (0, 0)),
        ],
        out_shape=[
            jax.ShapeDtypeStruct((B, N, cout), jnp.bfloat16),
            jax.ShapeDtypeStruct((2, cout), jnp.float32),
        ],
    )(x, wT)


def _mm2(xT, scale, bias, wT):
    cin, cout = wT.shape

    def kern(x_ref, s_ref, b_ref, w_ref, y_ref, stats_ref):
        x = jnp.maximum(
            x_ref[0].astype(jnp.float32) * s_ref[0][None, :] + b_ref[0][None, :],
            0.0)
        y = jnp.dot(x, w_ref[...], preferred_element_type=jnp.float32)
        y_ref[0] = y.astype(jnp.bfloat16).reshape(NB_MM, 16, 128)

        @pl.when((pl.program_id(0) == 0) & (pl.program_id(1) == 0))
        def _():
            stats_ref[...] = jnp.zeros_like(stats_ref)

        stats_ref[...] += jnp.concatenate(
            [jnp.sum(y, axis=0)[None, :], jnp.sum(y * y, axis=0)[None, :]],
            axis=0)

    return pl.pallas_call(
        kern,
        grid=(B, N // NB_MM),
        in_specs=[
            pl.BlockSpec((1, NB_MM, cin), lambda b, n: (b, n, 0)),
            pl.BlockSpec((1, cin), lambda b, n: (0, 0)),
            pl.BlockSpec((1, cin), lambda b, n: (0, 0)),
            pl.BlockSpec((cin, cout), lambda b, n: (0, 0)),
        ],
        out_specs=[
            pl.BlockSpec((1, NB_MM, 16, 128), lambda b, n: (b, n, 0, 0)),
            pl.BlockSpec((2, cout), lambda b, n: (0, 0)),
        ],
        out_shape=[
            jax.ShapeDtypeStruct((B, N, 16, 128), jnp.bfloat16),
            jax.ShapeDtypeStruct((2, cout), jnp.float32),
        ],
    )(xT, scale[None, :], bias[None, :], wT)


def _gather_max_kernel(knn_sm, y_ref, pe_ref, s2_ref, b2_ref, o_ref, g_ref):
    b = pl.program_id(0)
    nb = pl.program_id(1)
    base = b * (N * K) + nb * (NB_G * K)

    def body(j, carry):
        n = j & (NB_G - 1)
        k = j >> 7
        idx = knn_sm[base + n * K + k]
        g_ref[j] = y_ref[0, idx]
        return carry

    jax.lax.fori_loop(0, NB_G * K, body, 0, unroll=8)

    g2 = g_ref[...].reshape(NB_G * K, C2)
    gn = jnp.maximum(g2.T.astype(jnp.float32) * s2_ref[...] + b2_ref[...], 0.0)
    s = gn + pe_ref[0].reshape(C2, K * NB_G)
    m = s[:, 0:NB_G]
    for k in range(1, K):
        m = jnp.maximum(m, s[:, k * NB_G:(k + 1) * NB_G])
    o_ref[0] = m


def _gather_max(y2, pe_kn, knn_flat, s2, b2):
    grid_spec = pltpu.PrefetchScalarGridSpec(
        num_scalar_prefetch=1,
        grid=(B, N // NB_G),
        in_specs=[
            pl.BlockSpec((1, N, 16, 128), lambda b, n, knn: (b, 0, 0, 0)),
            pl.BlockSpec((1, C2, K, NB_G), lambda b, n, knn: (b, 0, 0, n)),
            pl.BlockSpec((C2, 1), lambda b, n, knn: (0, 0)),
            pl.BlockSpec((C2, 1), lambda b, n, knn: (0, 0)),
        ],
        out_specs=pl.BlockSpec((1, C2, NB_G), lambda b, n, knn: (b, 0, n)),
        scratch_shapes=[pltpu.VMEM((NB_G * K, 16, 128), jnp.bfloat16)],
    )
    return pl.pallas_call(
        _gather_max_kernel,
        grid_spec=grid_spec,
        out_shape=jax.ShapeDtypeStruct((B, C2, N), jnp.float32),
    )(knn_flat, y2, pe_kn, s2[:, None], b2[:, None])


def _bn_coeffs(stats, gamma, beta):
    cnt = float(B * N)
    mean = stats[0] / cnt
    var = stats[1] / cnt - mean * mean
    scale = gamma * jax.lax.rsqrt(var + EPS)
    bias = beta - mean * scale
    return scale, bias


def kernel(p, f, pe, knn_idx, W1, gamma1, beta1, W2, gamma2, beta2):
    y1, st1 = _mm1(f, W1.T)
    s1, b1 = _bn_coeffs(st1, gamma1, beta1)
    y2, st2 = _mm2(y1, s1, b1, W2.T)
    s2, b2 = _bn_coeffs(st2, gamma2, beta2)
    pe_kn = jnp.transpose(pe, (0, 1, 3, 2))
    out = _gather_max(y2, pe_kn, knn_idx.reshape(-1), s2, b2)
    return (out, knn_idx)


# speedup vs baseline: 4.1920x; 1.0142x over previous
import functools

import jax
import jax.numpy as jnp
from jax.experimental import pallas as pl
from jax.experimental.pallas import tpu as pltpu

B, N, K = 4, 1024, 8
C0, C1, C2 = 1024, 2048, 2048
EPS = 1e-5

NB_MM = 256
NB_G = 128


def _mm1_kernel(x_ref, w_ref, y_ref, stats_ref):
    y = jax.lax.dot_general(x_ref[0], w_ref[...], (((0,), (0,)), ((), ())),
                            preferred_element_type=jnp.float32)
    y_ref[0] = y.astype(jnp.bfloat16)

    @pl.when((pl.program_id(0) == 0) & (pl.program_id(1) == 0))
    def _():
        stats_ref[...] = jnp.zeros_like(stats_ref)

    stats_ref[...] += jnp.concatenate(
        [jnp.sum(y, axis=0)[None, :], jnp.sum(y * y, axis=0)[None, :]], axis=0)


def _mm1(x, wT):
    cin, cout = wT.shape
    return pl.pallas_call(
        _mm1_kernel,
        grid=(B, N // NB_MM),
        in_specs=[
            pl.BlockSpec((1, cin, NB_MM), lambda b, n: (b, 0, n)),
            pl.BlockSpec((cin, cout), lambda b, n: (0, 0)),
        ],
        out_specs=[
            pl.BlockSpec((1, NB_MM, cout), lambda b, n: (b, n, 0)),
            pl.BlockSpec((2, cout), lambda b, n: (0, 0)),
        ],
        out_shape=[
            jax.ShapeDtypeStruct((B, N, cout), jnp.bfloat16),
            jax.ShapeDtypeStruct((2, cout), jnp.float32),
        ],
    )(x, wT)


def _mm2(xT, scale, bias, wT):
    cin, cout = wT.shape

    def kern(x_ref, s_ref, b_ref, w_ref, y_ref, stats_ref):
        x = jnp.maximum(
            x_ref[0].astype(jnp.float32) * s_ref[0][None, :] + b_ref[0][None, :],
            0.0)
        y = jax.lax.dot_general(w_ref[...], x, (((0,), (1,)), ((), ())),
                                preferred_element_type=jnp.float32)
        y_ref[0] = y.astype(jnp.bfloat16)

        @pl.when((pl.program_id(0) == 0) & (pl.program_id(1) == 0))
        def _():
            stats_ref[...] = jnp.zeros_like(stats_ref)

        stats_ref[...] += jnp.concatenate(
            [jnp.sum(y, axis=1)[None, :], jnp.sum(y * y, axis=1)[None, :]],
            axis=0)

    return pl.pallas_call(
        kern,
        grid=(B, N // NB_MM),
        in_specs=[
            pl.BlockSpec((1, NB_MM, cin), lambda b, n: (b, n, 0)),
            pl.BlockSpec((1, cin), lambda b, n: (0, 0)),
            pl.BlockSpec((1, cin), lambda b, n: (0, 0)),
            pl.BlockSpec((cin, cout), lambda b, n: (0, 0)),
        ],
        out_specs=[
            pl.BlockSpec((1, cout, NB_MM), lambda b, n: (b, 0, n)),
            pl.BlockSpec((2, cout), lambda b, n: (0, 0)),
        ],
        out_shape=[
            jax.ShapeDtypeStruct((B, cout, N), jnp.bfloat16),
            jax.ShapeDtypeStruct((2, cout), jnp.float32),
        ],
    )(xT, scale[None, :], bias[None, :], wT)


def _norm_kernel(y_ref, s_ref, b_ref, o_ref):
    o_ref[0] = jnp.maximum(
        y_ref[0].astype(jnp.float32) * s_ref[...] + b_ref[...],
        0.0).astype(jnp.bfloat16)


def _normalize(y, scale, bias):
    return pl.pallas_call(
        _norm_kernel,
        grid=(B,),
        in_specs=[
            pl.BlockSpec((1, C2, N), lambda b: (b, 0, 0)),
            pl.BlockSpec((C2, 1), lambda b: (0, 0)),
            pl.BlockSpec((C2, 1), lambda b: (0, 0)),
        ],
        out_specs=pl.BlockSpec((1, C2, N), lambda b: (b, 0, 0)),
        out_shape=jax.ShapeDtypeStruct((B, C2, N), jnp.bfloat16),
    )(y, scale[:, None], bias[:, None])


def _gather_max_kernel(y_ref, knn_ref, pe_ref, o_ref):
    row = knn_ref[0].reshape(1, K * NB_G)
    iota = jax.lax.broadcasted_iota(jnp.int32, (N, K * NB_G), 0)
    oh = (iota == row).astype(jnp.bfloat16)
    g = jax.lax.dot_general(y_ref[0], oh, (((1,), (0,)), ((), ())),
                            preferred_element_type=jnp.float32)
    s = g + pe_ref[0].reshape(C2, K * NB_G)
    m = s[:, 0:NB_G]
    for k in range(1, K):
        m = jnp.maximum(m, s[:, k * NB_G:(k + 1) * NB_G])
    o_ref[0] = m


def _gather_max(y2n, knn_t, pe_kn):
    return pl.pallas_call(
        _gather_max_kernel,
        grid=(B, N // NB_G),
        in_specs=[
            pl.BlockSpec((1, C2, N), lambda b, n: (b, 0, 0)),
            pl.BlockSpec((1, K, NB_G), lambda b, n: (b, 0, n)),
            pl.BlockSpec((1, C2, K, NB_G), lambda b, n: (b, 0, 0, n)),
        ],
        out_specs=pl.BlockSpec((1, C2, NB_G), lambda b, n: (b, 0, n)),
        out_shape=jax.ShapeDtypeStruct((B, C2, N), jnp.float32),
    )(y2n, knn_t, pe_kn)


def _bn_coeffs(stats, gamma, beta):
    cnt = float(B * N)
    mean = stats[0] / cnt
    var = stats[1] / cnt - mean * mean
    scale = gamma * jax.lax.rsqrt(var + EPS)
    bias = beta - mean * scale
    return scale, bias


def kernel(p, f, pe, knn_idx, W1, gamma1, beta1, W2, gamma2, beta2):
    y1, st1 = _mm1(f, W1.T)
    s1, b1 = _bn_coeffs(st1, gamma1, beta1)
    y2, st2 = _mm2(y1, s1, b1, W2.T)
    s2, b2 = _bn_coeffs(st2, gamma2, beta2)
    y2n = _normalize(y2, s2, b2)
    pe_kn = jnp.transpose(pe, (0, 1, 3, 2))
    knn_t = jnp.transpose(knn_idx, (0, 2, 1))
    out = _gather_max(y2n, knn_t, pe_kn)
    return (out, knn_idx)


# speedup vs baseline: 4.8655x; 1.1607x over previous
import functools

import jax
import jax.numpy as jnp
from jax.experimental import pallas as pl
from jax.experimental.pallas import tpu as pltpu

B, N, K = 4, 1024, 8
C0, C1, C2 = 1024, 2048, 2048
EPS = 1e-5

NB_MM = 256
NB_G = 128


def _mm1_kernel(x_ref, w_ref, y_ref, stats_ref):
    y = jax.lax.dot_general(x_ref[0], w_ref[...], (((0,), (1,)), ((), ())),
                            preferred_element_type=jnp.float32)
    y_ref[0] = y.astype(jnp.bfloat16)

    @pl.when((pl.program_id(0) == 0) & (pl.program_id(1) == 0))
    def _():
        stats_ref[...] = jnp.zeros_like(stats_ref)

    stats_ref[...] += jnp.concatenate(
        [jnp.sum(y, axis=0)[None, :], jnp.sum(y * y, axis=0)[None, :]], axis=0)


def _mm1(x, w):
    cout, cin = w.shape
    return pl.pallas_call(
        _mm1_kernel,
        grid=(B, N // NB_MM),
        in_specs=[
            pl.BlockSpec((1, cin, NB_MM), lambda b, n: (b, 0, n)),
            pl.BlockSpec((cout, cin), lambda b, n: (0, 0)),
        ],
        out_specs=[
            pl.BlockSpec((1, NB_MM, cout), lambda b, n: (b, n, 0)),
            pl.BlockSpec((2, cout), lambda b, n: (0, 0)),
        ],
        out_shape=[
            jax.ShapeDtypeStruct((B, N, cout), jnp.bfloat16),
            jax.ShapeDtypeStruct((2, cout), jnp.float32),
        ],
    )(x, w)


def _mm2(xT, scale, bias, w):
    cout, cin = w.shape

    def kern(x_ref, s_ref, b_ref, w_ref, y_ref, stats_ref):
        x = jnp.maximum(
            x_ref[0].astype(jnp.float32) * s_ref[0][None, :] + b_ref[0][None, :],
            0.0)
        y = jax.lax.dot_general(w_ref[...], x, (((1,), (1,)), ((), ())),
                                preferred_element_type=jnp.float32)
        y_ref[0] = y.astype(jnp.bfloat16)

        @pl.when((pl.program_id(0) == 0) & (pl.program_id(1) == 0))
        def _():
            stats_ref[...] = jnp.zeros_like(stats_ref)

        stats_ref[...] += jnp.concatenate(
            [jnp.sum(y, axis=1)[None, :], jnp.sum(y * y, axis=1)[None, :]],
            axis=0)

    return pl.pallas_call(
        kern,
        grid=(B, N // NB_MM),
        in_specs=[
            pl.BlockSpec((1, NB_MM, cin), lambda b, n: (b, n, 0)),
            pl.BlockSpec((1, cin), lambda b, n: (0, 0)),
            pl.BlockSpec((1, cin), lambda b, n: (0, 0)),
            pl.BlockSpec((cout, cin), lambda b, n: (0, 0)),
        ],
        out_specs=[
            pl.BlockSpec((1, cout, NB_MM), lambda b, n: (b, 0, n)),
            pl.BlockSpec((2, cout), lambda b, n: (0, 0)),
        ],
        out_shape=[
            jax.ShapeDtypeStruct((B, cout, N), jnp.bfloat16),
            jax.ShapeDtypeStruct((2, cout), jnp.float32),
        ],
    )(xT, scale[None, :], bias[None, :], w)


def _norm_kernel(y_ref, s_ref, b_ref, o_ref):
    o_ref[0] = jnp.maximum(
        y_ref[0].astype(jnp.float32) * s_ref[...] + b_ref[...],
        0.0).astype(jnp.bfloat16)


def _normalize(y, scale, bias):
    return pl.pallas_call(
        _norm_kernel,
        grid=(B,),
        in_specs=[
            pl.BlockSpec((1, C2, N), lambda b: (b, 0, 0)),
            pl.BlockSpec((C2, 1), lambda b: (0, 0)),
            pl.BlockSpec((C2, 1), lambda b: (0, 0)),
        ],
        out_specs=pl.BlockSpec((1, C2, N), lambda b: (b, 0, 0)),
        out_shape=jax.ShapeDtypeStruct((B, C2, N), jnp.bfloat16),
    )(y, scale[:, None], bias[:, None])


def _gather_max_kernel(y_ref, knn_ref, pe_ref, o_ref):
    row = knn_ref[0].reshape(1, K * NB_G)
    iota = jax.lax.broadcasted_iota(jnp.int32, (N, K * NB_G), 0)
    oh = (iota == row).astype(jnp.bfloat16)
    g = jax.lax.dot_general(y_ref[0], oh, (((1,), (0,)), ((), ())),
                            preferred_element_type=jnp.float32)
    s = g + pe_ref[0].reshape(C2, K * NB_G)
    m = s[:, 0:NB_G]
    for k in range(1, K):
        m = jnp.maximum(m, s[:, k * NB_G:(k + 1) * NB_G])
    o_ref[0] = m


def _gather_max(y2n, knn_t, pe_kn):
    return pl.pallas_call(
        _gather_max_kernel,
        grid=(B, N // NB_G),
        in_specs=[
            pl.BlockSpec((1, C2, N), lambda b, n: (b, 0, 0)),
            pl.BlockSpec((1, K, NB_G), lambda b, n: (b, 0, n)),
            pl.BlockSpec((1, C2, K, NB_G), lambda b, n: (b, 0, 0, n)),
        ],
        out_specs=pl.BlockSpec((1, C2, NB_G), lambda b, n: (b, 0, n)),
        out_shape=jax.ShapeDtypeStruct((B, C2, N), jnp.float32),
    )(y2n, knn_t, pe_kn)


def _bn_coeffs(stats, gamma, beta):
    cnt = float(B * N)
    mean = stats[0] / cnt
    var = stats[1] / cnt - mean * mean
    scale = gamma * jax.lax.rsqrt(var + EPS)
    bias = beta - mean * scale
    return scale, bias


def kernel(p, f, pe, knn_idx, W1, gamma1, beta1, W2, gamma2, beta2):
    y1, st1 = _mm1(f, W1)
    s1, b1 = _bn_coeffs(st1, gamma1, beta1)
    y2, st2 = _mm2(y1, s1, b1, W2)
    s2, b2 = _bn_coeffs(st2, gamma2, beta2)
    y2n = _normalize(y2, s2, b2)
    pe_kn = jnp.transpose(pe, (0, 1, 3, 2))
    knn_t = jnp.transpose(knn_idx, (0, 2, 1))
    out = _gather_max(y2n, knn_t, pe_kn)
    return (out, knn_idx)
